# Initial kernel scaffold; baseline (speedup 1.0000x reference)
import jax
import jax.numpy as jnp
from jax.experimental import pallas as pl


def kernel(data_x, data_node_index, data_node_one_hot, n_id0, edge_index0, n_id1, edge_index1, emb_table, W1, b1, W2, b2, Wout, bout):
    raise NotImplementedError("write your pallas kernel here")


# speedup vs baseline: 5.6870x; 5.6870x over previous
import functools

import jax
import jax.numpy as jnp
from jax import lax
from jax.experimental import pallas as pl
from jax.experimental.pallas import tpu as pltpu
import jax.experimental.pallas.tpu_sc as plsc

_N = 10000
_NP = 10240
_E = 320000
_EP = 327680
_K = 128
_CPT = _EP // (16 * _K)
_RPT = _NP // 16
_RSUB = 128
_NSUB = _RPT // _RSUB
_GRP = 16
_H = 256
_C = 40
_MESH = plsc.VectorSubcoreMesh(core_axis_name="c", subcore_axis_name="s")
_SC_PARAMS = pltpu.CompilerParams(use_tc_tiling_on_sc=False)


def _conv1_body(dx0, dx1p, dnode, nidr, emb, srcr, dstr, ones8, z64, z8,
                o_acc, o_aux,
                xsh, acc_sh, asrc_sh, aacc_sh,
                nid2, t2, rows_v, rows8, sidx2, didx2, ones_v, sem):
    cid = lax.axis_index("c")
    sid = lax.axis_index("s")
    base = sid * _RPT

    pltpu.sync_copy(z64, rows_v)
    pltpu.sync_copy(z8, rows8)
    pltpu.sync_copy(ones8, ones_v)

    for i in range(_NSUB):
        r0 = base + i * _RSUB
        pltpu.sync_copy(rows_v, acc_sh.at[pl.ds(r0, _RSUB)])
        pltpu.sync_copy(rows8, aacc_sh.at[pl.ds(r0, _RSUB)])

    pltpu.sync_copy(nidr.at[sid], nid2)
    for i in range(_NSUB):
        r0 = base + i * _RSUB
        idx = nid2.at[i]

        @pl.when(cid == 0)
        def _():
            pltpu.async_copy(dx0.at[idx], rows_v, sem).wait()
            pltpu.sync_copy(rows_v, xsh.at[pl.ds(r0, _RSUB)])

        @pl.when(cid == 1)
        def _():
            pltpu.async_copy(dnode.at[idx], t2.at[i], sem).wait()
            pltpu.async_copy(emb.at[t2.at[i]], rows8, sem).wait()
            pltpu.sync_copy(rows8, asrc_sh.at[pl.ds(r0, _RSUB)])
            pltpu.async_copy(dx1p.at[idx], rows_v, sem).wait()
            pltpu.sync_copy(rows_v, xsh.at[pl.ds(r0, _RSUB)])

    plsc.subcore_barrier()

    def group_body(g, carry):
        w = sid * (_CPT // _GRP) + g
        pltpu.sync_copy(srcr.at[w], sidx2)
        pltpu.sync_copy(dstr.at[w], didx2)

        def edge_body(j, carry2):
            idxs = sidx2.at[j]
            idxd = didx2.at[j]
            pltpu.async_copy(xsh.at[idxs], rows_v, sem).wait()
            pltpu.sync_copy(rows_v, acc_sh.at[idxd], add=True)

            @pl.when(cid == 0)
            def _():
                pltpu.sync_copy(ones_v, aacc_sh.at[idxd], add=True)

            @pl.when(cid == 1)
            def _():
                pltpu.async_copy(asrc_sh.at[idxs], rows8, sem).wait()
                pltpu.sync_copy(rows8, aacc_sh.at[idxd], add=True)

            return carry2

        lax.fori_loop(0, _GRP, edge_body, 0)
        return carry

    lax.fori_loop(0, _CPT // _GRP, group_body, 0)

    plsc.subcore_barrier()

    for i in range(_NSUB):
        r0 = base + i * _RSUB
        pltpu.sync_copy(acc_sh.at[pl.ds(r0, _RSUB)], rows_v)
        pltpu.sync_copy(rows_v, o_acc.at[pl.ds(cid * _NP + r0, _RSUB)])
        pltpu.sync_copy(aacc_sh.at[pl.ds(r0, _RSUB)], rows8)
        pltpu.sync_copy(rows8, o_aux.at[pl.ds(cid * _NP + r0, _RSUB)])


_conv1 = functools.partial(
    pl.kernel,
    out_type=(
        jax.ShapeDtypeStruct((2 * _NP, 64), jnp.float32),
        jax.ShapeDtypeStruct((2 * _NP, 8), jnp.float32),
    ),
    mesh=_MESH,
    scratch_types=[
        pltpu.VMEM_SHARED((_NP, 64), jnp.float32),
        pltpu.VMEM_SHARED((_NP, 64), jnp.float32),
        pltpu.VMEM_SHARED((_NP, 8), jnp.float32),
        pltpu.VMEM_SHARED((_NP, 8), jnp.float32),
        pltpu.VMEM((_NSUB, _RSUB), jnp.int32),
        pltpu.VMEM((_NSUB, _RSUB), jnp.int32),
        pltpu.VMEM((_K, 64), jnp.float32),
        pltpu.VMEM((_K, 8), jnp.float32),
        pltpu.VMEM((_GRP, _K), jnp.int32),
        pltpu.VMEM((_GRP, _K), jnp.int32),
        pltpu.VMEM((_K, 8), jnp.float32),
        pltpu.SemaphoreType.DMA,
    ],
    compiler_params=_SC_PARAMS,
)(_conv1_body)


def _conv2_body(h1f, srcb, dstr, ones8, z128, z8,
                o_acc, o_aux,
                acc_sh, aacc_sh,
                rows_v, rows8, sidx2, didx2, ones_v, sem):
    cid = lax.axis_index("c")
    sid = lax.axis_index("s")
    base = sid * _RPT

    pltpu.sync_copy(z128, rows_v)
    pltpu.sync_copy(z8, rows8)
    pltpu.sync_copy(ones8, ones_v)

    for i in range(_NSUB):
        r0 = base + i * _RSUB
        pltpu.sync_copy(rows_v, acc_sh.at[pl.ds(r0, _RSUB)])
        pltpu.sync_copy(rows8, aacc_sh.at[pl.ds(r0, _RSUB)])

    def group_body(g, carry):
        w = sid * (_CPT // _GRP) + g
        pltpu.sync_copy(srcb.at[(cid * 16 + sid) * (_CPT // _GRP) + g], sidx2)
        pltpu.sync_copy(dstr.at[w], didx2)

        def edge_body(j, carry2):
            idxs = sidx2.at[j]
            idxd = didx2.at[j]
            pltpu.async_copy(h1f.at[idxs], rows_v, sem).wait()
            pltpu.sync_copy(rows_v, acc_sh.at[idxd], add=True)

            @pl.when(cid == 0)
            def _():
                pltpu.sync_copy(ones_v, aacc_sh.at[idxd], add=True)

            return carry2

        lax.fori_loop(0, _GRP, edge_body, 0)
        return carry

    lax.fori_loop(0, _CPT // _GRP, group_body, 0)

    plsc.subcore_barrier()

    for i in range(_NSUB):
        r0 = base + i * _RSUB
        pltpu.sync_copy(acc_sh.at[pl.ds(r0, _RSUB)], rows_v)
        pltpu.sync_copy(rows_v, o_acc.at[pl.ds(cid * _NP + r0, _RSUB)])

        @pl.when(cid == 0)
        def _():
            pltpu.sync_copy(aacc_sh.at[pl.ds(r0, _RSUB)], rows8)
            pltpu.sync_copy(rows8, o_aux.at[pl.ds(r0, _RSUB)])


_conv2 = functools.partial(
    pl.kernel,
    out_type=(
        jax.ShapeDtypeStruct((2 * _NP, 128), jnp.float32),
        jax.ShapeDtypeStruct((_NP, 8), jnp.float32),
    ),
    mesh=_MESH,
    scratch_types=[
        pltpu.VMEM_SHARED((_NP, 128), jnp.float32),
        pltpu.VMEM_SHARED((_NP, 8), jnp.float32),
        pltpu.VMEM((_K, 128), jnp.float32),
        pltpu.VMEM((_K, 8), jnp.float32),
        pltpu.VMEM((_GRP, _K), jnp.int32),
        pltpu.VMEM((_GRP, _K), jnp.int32),
        pltpu.VMEM((_K, 8), jnp.float32),
        pltpu.SemaphoreType.DMA,
    ],
    compiler_params=_SC_PARAMS,
)(_conv2_body)


def _mm1_body(agg_ref, aux_ref, w_ref, b_ref, o_ref):
    a0 = agg_ref[0]
    a1 = agg_ref[1]
    ae = aux_ref[1]
    cnt = aux_ref[0, :, 0:1]
    inv = 1.0 / jnp.maximum(cnt, 1.0)
    x = jnp.concatenate([ae, a0, a1[:, 0:56]], axis=1) * inv
    h = jnp.dot(x, w_ref[...], preferred_element_type=jnp.float32) + b_ref[...]
    h = jnp.maximum(h, 0.0)
    o_ref[0] = h[:, 0:128]
    o_ref[1] = h[:, 128:256]


def _mm2_body(agg_ref, cnt_ref, w2_ref, b2_ref, wo_ref, bo_ref, o_ref):
    cnt = cnt_ref[:, 0:1]
    inv = 1.0 / jnp.maximum(cnt, 1.0)
    x = jnp.concatenate([agg_ref[0], agg_ref[1]], axis=1) * inv
    h = jnp.dot(x, w2_ref[...], preferred_element_type=jnp.float32) + b2_ref[...]
    h = jnp.maximum(h, 0.0)
    lg = jnp.dot(h, wo_ref[...], preferred_element_type=jnp.float32) + bo_ref[...]
    m = jnp.max(lg, axis=1, keepdims=True)
    e = jnp.exp(lg - m)
    s = jnp.sum(e, axis=1, keepdims=True)
    o_ref[...] = lg - m - jnp.log(s)


_BN = 1000


def _mm1(agg, aux, w1, b1):
    return pl.pallas_call(
        _mm1_body,
        grid=(_N // _BN,),
        in_specs=[
            pl.BlockSpec((2, _BN, 64), lambda i: (0, i, 0)),
            pl.BlockSpec((2, _BN, 8), lambda i: (0, i, 0)),
            pl.BlockSpec((128, _H), lambda i: (0, 0)),
            pl.BlockSpec((1, _H), lambda i: (0, 0)),
        ],
        out_specs=pl.BlockSpec((2, _BN, 128), lambda i: (0, i, 0)),
        out_shape=jax.ShapeDtypeStruct((2, _N, 128), jnp.float32),
    )(agg, aux, w1, b1)


def _mm2(agg, cnt, w2, b2, wo, bo):
    return pl.pallas_call(
        _mm2_body,
        grid=(_N // _BN,),
        in_specs=[
            pl.BlockSpec((2, _BN, 128), lambda i: (0, i, 0)),
            pl.BlockSpec((_BN, 8), lambda i: (i, 0)),
            pl.BlockSpec((_H, _H), lambda i: (0, 0)),
            pl.BlockSpec((1, _H), lambda i: (0, 0)),
            pl.BlockSpec((_H, _C), lambda i: (0, 0)),
            pl.BlockSpec((1, _C), lambda i: (0, 0)),
        ],
        out_specs=pl.BlockSpec((_BN, _C), lambda i: (i, 0)),
        out_shape=jax.ShapeDtypeStruct((_N, _C), jnp.float32),
    )(agg, cnt, w2, b2, wo, bo)


def kernel(data_x, data_node_index, data_node_one_hot, n_id0, edge_index0,
           n_id1, edge_index1, emb_table, W1, b1, W2, b2, Wout, bout):
    del data_node_one_hot, n_id1
    dx0 = data_x[:, 0:64]
    dx1p = jnp.concatenate(
        [data_x[:, 64:120], jnp.zeros((_N, 8), jnp.float32)], axis=1)
    nidr = jnp.concatenate(
        [n_id0, jnp.zeros((_NP - _N,), jnp.int32)]).reshape(16, _NSUB, _RSUB)

    pidx = jnp.arange(_EP - _E, dtype=jnp.int32)
    ps = pidx % _N
    pd = _N + pidx % (_NP - _N)
    src0r = jnp.concatenate([edge_index0[0], ps]).reshape(-1, _GRP, _K)
    dst0r = jnp.concatenate([edge_index0[1], pd]).reshape(-1, _GRP, _K)
    s1p = jnp.concatenate([edge_index1[0], ps])
    srcb = jnp.concatenate([s1p, s1p + _N]).reshape(-1, _GRP, _K)
    dst1r = jnp.concatenate([edge_index1[1], pd]).reshape(-1, _GRP, _K)

    ones8 = jnp.ones((_K, 8), jnp.float32)
    z64 = jnp.zeros((_RSUB, 64), jnp.float32)
    z8 = jnp.zeros((_RSUB, 8), jnp.float32)
    z128 = jnp.zeros((_RSUB, 128), jnp.float32)

    o_acc, o_aux = _conv1(dx0, dx1p, data_node_index, nidr, emb_table,
                          src0r, dst0r, ones8, z64, z8)
    h1s = _mm1(o_acc.reshape(2, _NP, 64), o_aux.reshape(2, _NP, 8),
               W1, b1.reshape(1, _H))
    o3, o4 = _conv2(h1s.reshape(2 * _N, 128), srcb, dst1r, ones8, z128, z8)
    return _mm2(o3.reshape(2, _NP, 128), o4, W2, b2.reshape(1, _H),
                Wout, bout.reshape(1, _C))


# speedup vs baseline: 7.5640x; 1.3300x over previous
import functools

import jax
import jax.numpy as jnp
from jax import lax
from jax.experimental import pallas as pl
from jax.experimental.pallas import tpu as pltpu
import jax.experimental.pallas.tpu_sc as plsc

_N = 10000
_NP = 10240
_E = 320000
_EP = 327680
_K = 128
_CPT = _EP // (16 * _K)
_RPT = _NP // 16
_RSUB = 128
_NSUB = _RPT // _RSUB
_GRP = 16
_H = 256
_C = 40
_MESH = plsc.VectorSubcoreMesh(core_axis_name="c", subcore_axis_name="s")
_SC_PARAMS = pltpu.CompilerParams(use_tc_tiling_on_sc=False)


def _conv1_body(dx0, dx1p, dnode, nidr, emb, srcr, dstr, ones8, z64, z8,
                o_acc, o_aux,
                xsh, acc_sh, asrc_sh, aacc_sh,
                nid2, t2, rows_v, rows_b, rows8, rows8_b,
                sidx2, didx2, ones_v, sems):
    sem = sems.at[0]
    cid = lax.axis_index("c")
    sid = lax.axis_index("s")
    base = sid * _RPT

    pltpu.sync_copy(z64, rows_v)
    pltpu.sync_copy(z8, rows8)
    pltpu.sync_copy(ones8, ones_v)

    for i in range(_NSUB):
        r0 = base + i * _RSUB
        pltpu.sync_copy(rows_v, acc_sh.at[pl.ds(r0, _RSUB)])
        pltpu.sync_copy(rows8, aacc_sh.at[pl.ds(r0, _RSUB)])

    pltpu.sync_copy(nidr.at[sid], nid2)
    for i in range(_NSUB):
        r0 = base + i * _RSUB
        idx = nid2.at[i]

        @pl.when(cid == 0)
        def _():
            pltpu.async_copy(dx0.at[idx], rows_v, sem).wait()
            pltpu.sync_copy(rows_v, xsh.at[pl.ds(r0, _RSUB)])

        @pl.when(cid == 1)
        def _():
            pltpu.async_copy(dnode.at[idx], t2.at[i], sem).wait()
            pltpu.async_copy(emb.at[t2.at[i]], rows8, sem).wait()
            pltpu.sync_copy(rows8, asrc_sh.at[pl.ds(r0, _RSUB)])
            pltpu.async_copy(dx1p.at[idx], rows_v, sem).wait()
            pltpu.sync_copy(rows_v, xsh.at[pl.ds(r0, _RSUB)])

    plsc.subcore_barrier()

    bufs = (rows_v, rows_b)
    b8s = (rows8, rows8_b)
    gsem = (sems.at[0], sems.at[1])
    ssem = (sems.at[2], sems.at[3])
    g8sem = (sems.at[4], sems.at[5])
    s8sem = (sems.at[6], sems.at[7])
    sem1 = sems.at[8]

    def group_body(g, carry):
        w = sid * (_CPT // _GRP) + g
        pltpu.sync_copy(srcr.at[w], sidx2)
        pltpu.sync_copy(dstr.at[w], didx2)

        pltpu.async_copy(xsh.at[sidx2.at[0]], bufs[0], gsem[0])

        @pl.when(cid == 1)
        def _():
            pltpu.async_copy(asrc_sh.at[sidx2.at[0]], b8s[0], g8sem[0])

        for j in range(_GRP):
            p = j & 1
            q = 1 - p
            idxs = sidx2.at[j]
            idxd = didx2.at[j]
            if j + 1 < _GRP:
                idxn = sidx2.at[j + 1]
                pltpu.async_copy(xsh.at[idxn], bufs[q], gsem[q])

                @pl.when(cid == 1)
                def _():
                    pltpu.async_copy(asrc_sh.at[idxn], b8s[q], g8sem[q])

            pltpu.make_async_copy(xsh.at[idxs], bufs[p], gsem[p]).wait()
            pltpu.sync_copy(bufs[p], acc_sh.at[idxd], add=True)

            @pl.when(cid == 1)
            def _():
                pltpu.make_async_copy(asrc_sh.at[idxs], b8s[p], g8sem[p]).wait()
                pltpu.sync_copy(b8s[p], aacc_sh.at[idxd], add=True)

            @pl.when(cid == 0)
            def _():
                pltpu.sync_copy(ones_v, aacc_sh.at[idxd], add=True)

        return carry

    lax.fori_loop(0, _CPT // _GRP, group_body, 0)

    plsc.subcore_barrier()

    for i in range(_NSUB):
        r0 = base + i * _RSUB
        pltpu.sync_copy(acc_sh.at[pl.ds(r0, _RSUB)], rows_v)
        pltpu.sync_copy(rows_v, o_acc.at[pl.ds(cid * _NP + r0, _RSUB)])
        pltpu.sync_copy(aacc_sh.at[pl.ds(r0, _RSUB)], rows8)
        pltpu.sync_copy(rows8, o_aux.at[pl.ds(cid * _NP + r0, _RSUB)])


_conv1 = functools.partial(
    pl.kernel,
    out_type=(
        jax.ShapeDtypeStruct((2 * _NP, 64), jnp.float32),
        jax.ShapeDtypeStruct((2 * _NP, 8), jnp.float32),
    ),
    mesh=_MESH,
    scratch_types=[
        pltpu.VMEM_SHARED((_NP, 64), jnp.float32),
        pltpu.VMEM_SHARED((_NP, 64), jnp.float32),
        pltpu.VMEM_SHARED((_NP, 8), jnp.float32),
        pltpu.VMEM_SHARED((_NP, 8), jnp.float32),
        pltpu.VMEM((_NSUB, _RSUB), jnp.int32),
        pltpu.VMEM((_NSUB, _RSUB), jnp.int32),
        pltpu.VMEM((_K, 64), jnp.float32),
        pltpu.VMEM((_K, 64), jnp.float32),
        pltpu.VMEM((_K, 8), jnp.float32),
        pltpu.VMEM((_K, 8), jnp.float32),
        pltpu.VMEM((_GRP, _K), jnp.int32),
        pltpu.VMEM((_GRP, _K), jnp.int32),
        pltpu.VMEM((_K, 8), jnp.float32),
        pltpu.SemaphoreType.DMA((9,)),
    ],
    compiler_params=_SC_PARAMS,
)(_conv1_body)


def _conv2_body(h1f, srcb, dstr, ones8, z128, z8,
                o_acc, o_aux,
                acc_sh, aacc_sh,
                rows_v, rows_b, rows8, sidx2, didx2, ones_v, sems):
    cid = lax.axis_index("c")
    sid = lax.axis_index("s")
    base = sid * _RPT

    pltpu.sync_copy(z128, rows_v)
    pltpu.sync_copy(z8, rows8)
    pltpu.sync_copy(ones8, ones_v)

    for i in range(_NSUB):
        r0 = base + i * _RSUB
        pltpu.sync_copy(rows_v, acc_sh.at[pl.ds(r0, _RSUB)])
        pltpu.sync_copy(rows8, aacc_sh.at[pl.ds(r0, _RSUB)])

    bufs = (rows_v, rows_b)
    gsem = (sems.at[0], sems.at[1])
    ssem = (sems.at[2], sems.at[3])
    sem1 = sems.at[4]

    def group_body(g, carry):
        w = sid * (_CPT // _GRP) + g
        pltpu.sync_copy(srcb.at[(cid * 16 + sid) * (_CPT // _GRP) + g], sidx2)
        pltpu.sync_copy(dstr.at[w], didx2)

        pltpu.async_copy(h1f.at[sidx2.at[0]], bufs[0], gsem[0])

        for j in range(_GRP):
            p = j & 1
            q = 1 - p
            idxs = sidx2.at[j]
            idxd = didx2.at[j]
            if j + 1 < _GRP:
                pltpu.async_copy(h1f.at[sidx2.at[j + 1]], bufs[q], gsem[q])

            pltpu.make_async_copy(h1f.at[idxs], bufs[p], gsem[p]).wait()
            pltpu.sync_copy(bufs[p], acc_sh.at[idxd], add=True)

            @pl.when(cid == 0)
            def _():
                pltpu.sync_copy(ones_v, aacc_sh.at[idxd], add=True)

        return carry

    lax.fori_loop(0, _CPT // _GRP, group_body, 0)

    plsc.subcore_barrier()

    for i in range(_NSUB):
        r0 = base + i * _RSUB
        pltpu.sync_copy(acc_sh.at[pl.ds(r0, _RSUB)], rows_v)
        pltpu.sync_copy(rows_v, o_acc.at[pl.ds(cid * _NP + r0, _RSUB)])

        @pl.when(cid == 0)
        def _():
            pltpu.sync_copy(aacc_sh.at[pl.ds(r0, _RSUB)], rows8)
            pltpu.sync_copy(rows8, o_aux.at[pl.ds(r0, _RSUB)])


_conv2 = functools.partial(
    pl.kernel,
    out_type=(
        jax.ShapeDtypeStruct((2 * _NP, 128), jnp.float32),
        jax.ShapeDtypeStruct((_NP, 8), jnp.float32),
    ),
    mesh=_MESH,
    scratch_types=[
        pltpu.VMEM_SHARED((_NP, 128), jnp.float32),
        pltpu.VMEM_SHARED((_NP, 8), jnp.float32),
        pltpu.VMEM((_K, 128), jnp.float32),
        pltpu.VMEM((_K, 128), jnp.float32),
        pltpu.VMEM((_K, 8), jnp.float32),
        pltpu.VMEM((_GRP, _K), jnp.int32),
        pltpu.VMEM((_GRP, _K), jnp.int32),
        pltpu.VMEM((_K, 8), jnp.float32),
        pltpu.SemaphoreType.DMA((5,)),
    ],
    compiler_params=_SC_PARAMS,
)(_conv2_body)


def _mm1_body(agg_ref, aux_ref, w_ref, b_ref, o_ref):
    a0 = agg_ref[0]
    a1 = agg_ref[1]
    ae = aux_ref[1]
    cnt = aux_ref[0, :, 0:1]
    inv = 1.0 / jnp.maximum(cnt, 1.0)
    x = jnp.concatenate([ae, a0, a1[:, 0:56]], axis=1) * inv
    h = jnp.dot(x, w_ref[...], preferred_element_type=jnp.float32) + b_ref[...]
    h = jnp.maximum(h, 0.0)
    o_ref[0] = h[:, 0:128]
    o_ref[1] = h[:, 128:256]


def _mm2_body(agg_ref, cnt_ref, w2_ref, b2_ref, wo_ref, bo_ref, o_ref):
    cnt = cnt_ref[:, 0:1]
    inv = 1.0 / jnp.maximum(cnt, 1.0)
    x = jnp.concatenate([agg_ref[0], agg_ref[1]], axis=1) * inv
    h = jnp.dot(x, w2_ref[...], preferred_element_type=jnp.float32) + b2_ref[...]
    h = jnp.maximum(h, 0.0)
    lg = jnp.dot(h, wo_ref[...], preferred_element_type=jnp.float32) + bo_ref[...]
    m = jnp.max(lg, axis=1, keepdims=True)
    e = jnp.exp(lg - m)
    s = jnp.sum(e, axis=1, keepdims=True)
    o_ref[...] = lg - m - jnp.log(s)


_BN = 1000


def _mm1(agg, aux, w1, b1):
    return pl.pallas_call(
        _mm1_body,
        grid=(_N // _BN,),
        in_specs=[
            pl.BlockSpec((2, _BN, 64), lambda i: (0, i, 0)),
            pl.BlockSpec((2, _BN, 8), lambda i: (0, i, 0)),
            pl.BlockSpec((128, _H), lambda i: (0, 0)),
            pl.BlockSpec((1, _H), lambda i: (0, 0)),
        ],
        out_specs=pl.BlockSpec((2, _BN, 128), lambda i: (0, i, 0)),
        out_shape=jax.ShapeDtypeStruct((2, _N, 128), jnp.float32),
    )(agg, aux, w1, b1)


def _mm2(agg, cnt, w2, b2, wo, bo):
    return pl.pallas_call(
        _mm2_body,
        grid=(_N // _BN,),
        in_specs=[
            pl.BlockSpec((2, _BN, 128), lambda i: (0, i, 0)),
            pl.BlockSpec((_BN, 8), lambda i: (i, 0)),
            pl.BlockSpec((_H, _H), lambda i: (0, 0)),
            pl.BlockSpec((1, _H), lambda i: (0, 0)),
            pl.BlockSpec((_H, _C), lambda i: (0, 0)),
            pl.BlockSpec((1, _C), lambda i: (0, 0)),
        ],
        out_specs=pl.BlockSpec((_BN, _C), lambda i: (i, 0)),
        out_shape=jax.ShapeDtypeStruct((_N, _C), jnp.float32),
    )(agg, cnt, w2, b2, wo, bo)


def kernel(data_x, data_node_index, data_node_one_hot, n_id0, edge_index0,
           n_id1, edge_index1, emb_table, W1, b1, W2, b2, Wout, bout):
    del data_node_one_hot, n_id1
    dx0 = data_x[:, 0:64]
    dx1p = jnp.concatenate(
        [data_x[:, 64:120], jnp.zeros((_N, 8), jnp.float32)], axis=1)
    nidr = jnp.concatenate(
        [n_id0, jnp.zeros((_NP - _N,), jnp.int32)]).reshape(16, _NSUB, _RSUB)

    pidx = jnp.arange(_EP - _E, dtype=jnp.int32)
    ps = pidx % _N
    pd = _N + pidx % (_NP - _N)
    src0r = jnp.concatenate([edge_index0[0], ps]).reshape(-1, _GRP, _K)
    dst0r = jnp.concatenate([edge_index0[1], pd]).reshape(-1, _GRP, _K)
    s1p = jnp.concatenate([edge_index1[0], ps])
    srcb = jnp.concatenate([s1p, s1p + _N]).reshape(-1, _GRP, _K)
    dst1r = jnp.concatenate([edge_index1[1], pd]).reshape(-1, _GRP, _K)

    ones8 = jnp.ones((_K, 8), jnp.float32)
    z64 = jnp.zeros((_RSUB, 64), jnp.float32)
    z8 = jnp.zeros((_RSUB, 8), jnp.float32)
    z128 = jnp.zeros((_RSUB, 128), jnp.float32)

    o_acc, o_aux = _conv1(dx0, dx1p, data_node_index, nidr, emb_table,
                          src0r, dst0r, ones8, z64, z8)
    h1s = _mm1(o_acc.reshape(2, _NP, 64), o_aux.reshape(2, _NP, 8),
               W1, b1.reshape(1, _H))
    o3, o4 = _conv2(h1s.reshape(2 * _N, 128), srcb, dst1r, ones8, z128, z8)
    return _mm2(o3.reshape(2, _NP, 128), o4, W2, b2.reshape(1, _H),
                Wout, bout.reshape(1, _C))


# speedup vs baseline: 7.7134x; 1.0198x over previous
import functools

import jax
import jax.numpy as jnp
from jax import lax
from jax.experimental import pallas as pl
from jax.experimental.pallas import tpu as pltpu
import jax.experimental.pallas.tpu_sc as plsc

_N = 10000
_NP = 10240
_E = 320000
_EP = 327680
_K = 128
_CPT = _EP // (16 * _K)
_RPT = _NP // 16
_RSUB = 128
_NSUB = _RPT // _RSUB
_GRP = 16
_H = 256
_C = 40
_MESH = plsc.VectorSubcoreMesh(core_axis_name="c", subcore_axis_name="s")
_SC_PARAMS = pltpu.CompilerParams(use_tc_tiling_on_sc=False)


def _conv1_body(dx0, dx1p, dnode, nidr, emb, srcr, dstr, ones8, z64, z8,
                o_acc, o_aux,
                xsh, acc_sh, asrc_sh, aacc_sh,
                nid2, t2, rows_v, rows_b, rows8, rows8_b,
                sidx2, didx2, ones_v, sems):
    sem = sems.at[0]
    cid = lax.axis_index("c")
    sid = lax.axis_index("s")
    base = sid * _RPT

    pltpu.sync_copy(z64, rows_v)
    pltpu.sync_copy(z8, rows8)
    pltpu.sync_copy(ones8, ones_v)

    for i in range(_NSUB):
        r0 = base + i * _RSUB
        pltpu.sync_copy(rows_v, acc_sh.at[pl.ds(r0, _RSUB)])
        pltpu.sync_copy(rows8, aacc_sh.at[pl.ds(r0, _RSUB)])

    pltpu.sync_copy(nidr.at[sid], nid2)
    for i in range(_NSUB):
        r0 = base + i * _RSUB
        idx = nid2.at[i]

        @pl.when(cid == 0)
        def _():
            pltpu.async_copy(dx0.at[idx], rows_v, sem).wait()
            pltpu.sync_copy(rows_v, xsh.at[pl.ds(r0, _RSUB)])

        @pl.when(cid == 1)
        def _():
            pltpu.async_copy(dnode.at[idx], t2.at[i], sem).wait()
            pltpu.async_copy(emb.at[t2.at[i]], rows8, sem).wait()
            pltpu.sync_copy(rows8, asrc_sh.at[pl.ds(r0, _RSUB)])
            pltpu.async_copy(dx1p.at[idx], rows_v, sem).wait()
            pltpu.sync_copy(rows_v, xsh.at[pl.ds(r0, _RSUB)])

    plsc.subcore_barrier()

    bufs = (rows_v, rows_b)
    b8s = (rows8, rows8_b)
    gsem = (sems.at[0], sems.at[1])
    ssem = (sems.at[2], sems.at[3])
    g8sem = (sems.at[4], sems.at[5])
    s8sem = (sems.at[6], sems.at[7])
    sem1 = sems.at[8]

    def group_body(g, carry):
        w = sid * (_CPT // _GRP) + g
        pltpu.sync_copy(srcr.at[w], sidx2)
        pltpu.sync_copy(dstr.at[w], didx2)

        pltpu.async_copy(xsh.at[sidx2.at[0]], bufs[0], gsem[0])

        @pl.when(cid == 1)
        def _():
            pltpu.async_copy(asrc_sh.at[sidx2.at[0]], b8s[0], g8sem[0])

        sdesc = [None, None]
        for j in range(_GRP):
            p = j & 1
            q = 1 - p
            idxs = sidx2.at[j]
            idxd = didx2.at[j]
            if j + 1 < _GRP:
                if j >= 1:
                    sdesc[q].wait()
                idxn = sidx2.at[j + 1]
                pltpu.async_copy(xsh.at[idxn], bufs[q], gsem[q])

                @pl.when(cid == 1)
                def _():
                    pltpu.async_copy(asrc_sh.at[idxn], b8s[q], g8sem[q])

            pltpu.make_async_copy(xsh.at[idxs], bufs[p], gsem[p]).wait()
            sdesc[p] = pltpu.async_copy(bufs[p], acc_sh.at[idxd], ssem[p],
                                        add=True)

            @pl.when(cid == 1)
            def _():
                pltpu.make_async_copy(asrc_sh.at[idxs], b8s[p], g8sem[p]).wait()
                pltpu.sync_copy(b8s[p], aacc_sh.at[idxd], add=True)

            @pl.when(cid == 0)
            def _():
                pltpu.sync_copy(ones_v, aacc_sh.at[idxd], add=True)

        sdesc[_GRP & 1].wait()
        sdesc[(_GRP - 1) & 1].wait()
        return carry

    lax.fori_loop(0, _CPT // _GRP, group_body, 0)

    plsc.subcore_barrier()

    for i in range(_NSUB):
        r0 = base + i * _RSUB
        pltpu.sync_copy(acc_sh.at[pl.ds(r0, _RSUB)], rows_v)
        pltpu.sync_copy(rows_v, o_acc.at[pl.ds(cid * _NP + r0, _RSUB)])
        pltpu.sync_copy(aacc_sh.at[pl.ds(r0, _RSUB)], rows8)
        pltpu.sync_copy(rows8, o_aux.at[pl.ds(cid * _NP + r0, _RSUB)])


_conv1 = functools.partial(
    pl.kernel,
    out_type=(
        jax.ShapeDtypeStruct((2 * _NP, 64), jnp.float32),
        jax.ShapeDtypeStruct((2 * _NP, 8), jnp.float32),
    ),
    mesh=_MESH,
    scratch_types=[
        pltpu.VMEM_SHARED((_NP, 64), jnp.float32),
        pltpu.VMEM_SHARED((_NP, 64), jnp.float32),
        pltpu.VMEM_SHARED((_NP, 8), jnp.float32),
        pltpu.VMEM_SHARED((_NP, 8), jnp.float32),
        pltpu.VMEM((_NSUB, _RSUB), jnp.int32),
        pltpu.VMEM((_NSUB, _RSUB), jnp.int32),
        pltpu.VMEM((_K, 64), jnp.float32),
        pltpu.VMEM((_K, 64), jnp.float32),
        pltpu.VMEM((_K, 8), jnp.float32),
        pltpu.VMEM((_K, 8), jnp.float32),
        pltpu.VMEM((_GRP, _K), jnp.int32),
        pltpu.VMEM((_GRP, _K), jnp.int32),
        pltpu.VMEM((_K, 8), jnp.float32),
        pltpu.SemaphoreType.DMA((9,)),
    ],
    compiler_params=_SC_PARAMS,
)(_conv1_body)


def _conv2_body(h1f, srcb, dstr, ones8, z128, z8,
                o_acc, o_aux,
                acc_sh, aacc_sh,
                rows_v, rows_b, rows8, sidx2, didx2, ones_v, sems):
    cid = lax.axis_index("c")
    sid = lax.axis_index("s")
    base = sid * _RPT

    pltpu.sync_copy(z128, rows_v)
    pltpu.sync_copy(z8, rows8)
    pltpu.sync_copy(ones8, ones_v)

    for i in range(_NSUB):
        r0 = base + i * _RSUB
        pltpu.sync_copy(rows_v, acc_sh.at[pl.ds(r0, _RSUB)])
        pltpu.sync_copy(rows8, aacc_sh.at[pl.ds(r0, _RSUB)])

    bufs = (rows_v, rows_b)
    gsem = (sems.at[0], sems.at[1])
    ssem = (sems.at[2], sems.at[3])
    sem1 = sems.at[4]

    def group_body(g, carry):
        w = sid * (_CPT // _GRP) + g
        pltpu.sync_copy(srcb.at[(cid * 16 + sid) * (_CPT // _GRP) + g], sidx2)
        pltpu.sync_copy(dstr.at[w], didx2)

        pltpu.async_copy(h1f.at[sidx2.at[0]], bufs[0], gsem[0])

        sdesc = [None, None]
        for j in range(_GRP):
            p = j & 1
            q = 1 - p
            idxs = sidx2.at[j]
            idxd = didx2.at[j]
            if j + 1 < _GRP:
                if j >= 1:
                    sdesc[q].wait()
                pltpu.async_copy(h1f.at[sidx2.at[j + 1]], bufs[q], gsem[q])

            pltpu.make_async_copy(h1f.at[idxs], bufs[p], gsem[p]).wait()
            sdesc[p] = pltpu.async_copy(bufs[p], acc_sh.at[idxd], ssem[p],
                                        add=True)

            @pl.when(cid == 0)
            def _():
                pltpu.sync_copy(ones_v, aacc_sh.at[idxd], add=True)

        sdesc[_GRP & 1].wait()
        sdesc[(_GRP - 1) & 1].wait()
        return carry

    lax.fori_loop(0, _CPT // _GRP, group_body, 0)

    plsc.subcore_barrier()

    for i in range(_NSUB):
        r0 = base + i * _RSUB
        pltpu.sync_copy(acc_sh.at[pl.ds(r0, _RSUB)], rows_v)
        pltpu.sync_copy(rows_v, o_acc.at[pl.ds(cid * _NP + r0, _RSUB)])

        @pl.when(cid == 0)
        def _():
            pltpu.sync_copy(aacc_sh.at[pl.ds(r0, _RSUB)], rows8)
            pltpu.sync_copy(rows8, o_aux.at[pl.ds(r0, _RSUB)])


_conv2 = functools.partial(
    pl.kernel,
    out_type=(
        jax.ShapeDtypeStruct((2 * _NP, 128), jnp.float32),
        jax.ShapeDtypeStruct((_NP, 8), jnp.float32),
    ),
    mesh=_MESH,
    scratch_types=[
        pltpu.VMEM_SHARED((_NP, 128), jnp.float32),
        pltpu.VMEM_SHARED((_NP, 8), jnp.float32),
        pltpu.VMEM((_K, 128), jnp.float32),
        pltpu.VMEM((_K, 128), jnp.float32),
        pltpu.VMEM((_K, 8), jnp.float32),
        pltpu.VMEM((_GRP, _K), jnp.int32),
        pltpu.VMEM((_GRP, _K), jnp.int32),
        pltpu.VMEM((_K, 8), jnp.float32),
        pltpu.SemaphoreType.DMA((5,)),
    ],
    compiler_params=_SC_PARAMS,
)(_conv2_body)


def _mm1_body(agg_ref, aux_ref, w_ref, b_ref, o_ref):
    a0 = agg_ref[0]
    a1 = agg_ref[1]
    ae = aux_ref[1]
    cnt = aux_ref[0, :, 0:1]
    inv = 1.0 / jnp.maximum(cnt, 1.0)
    x = jnp.concatenate([ae, a0, a1[:, 0:56]], axis=1) * inv
    h = jnp.dot(x, w_ref[...], preferred_element_type=jnp.float32) + b_ref[...]
    h = jnp.maximum(h, 0.0)
    o_ref[0] = h[:, 0:128]
    o_ref[1] = h[:, 128:256]


def _mm2_body(agg_ref, cnt_ref, w2_ref, b2_ref, wo_ref, bo_ref, o_ref):
    cnt = cnt_ref[:, 0:1]
    inv = 1.0 / jnp.maximum(cnt, 1.0)
    x = jnp.concatenate([agg_ref[0], agg_ref[1]], axis=1) * inv
    h = jnp.dot(x, w2_ref[...], preferred_element_type=jnp.float32) + b2_ref[...]
    h = jnp.maximum(h, 0.0)
    lg = jnp.dot(h, wo_ref[...], preferred_element_type=jnp.float32) + bo_ref[...]
    m = jnp.max(lg, axis=1, keepdims=True)
    e = jnp.exp(lg - m)
    s = jnp.sum(e, axis=1, keepdims=True)
    o_ref[...] = lg - m - jnp.log(s)


_BN = 1000


def _mm1(agg, aux, w1, b1):
    return pl.pallas_call(
        _mm1_body,
        grid=(_N // _BN,),
        in_specs=[
            pl.BlockSpec((2, _BN, 64), lambda i: (0, i, 0)),
            pl.BlockSpec((2, _BN, 8), lambda i: (0, i, 0)),
            pl.BlockSpec((128, _H), lambda i: (0, 0)),
            pl.BlockSpec((1, _H), lambda i: (0, 0)),
        ],
        out_specs=pl.BlockSpec((2, _BN, 128), lambda i: (0, i, 0)),
        out_shape=jax.ShapeDtypeStruct((2, _N, 128), jnp.float32),
    )(agg, aux, w1, b1)


def _mm2(agg, cnt, w2, b2, wo, bo):
    return pl.pallas_call(
        _mm2_body,
        grid=(_N // _BN,),
        in_specs=[
            pl.BlockSpec((2, _BN, 128), lambda i: (0, i, 0)),
            pl.BlockSpec((_BN, 8), lambda i: (i, 0)),
            pl.BlockSpec((_H, _H), lambda i: (0, 0)),
            pl.BlockSpec((1, _H), lambda i: (0, 0)),
            pl.BlockSpec((_H, _C), lambda i: (0, 0)),
            pl.BlockSpec((1, _C), lambda i: (0, 0)),
        ],
        out_specs=pl.BlockSpec((_BN, _C), lambda i: (i, 0)),
        out_shape=jax.ShapeDtypeStruct((_N, _C), jnp.float32),
    )(agg, cnt, w2, b2, wo, bo)


def kernel(data_x, data_node_index, data_node_one_hot, n_id0, edge_index0,
           n_id1, edge_index1, emb_table, W1, b1, W2, b2, Wout, bout):
    del data_node_one_hot, n_id1
    dx0 = data_x[:, 0:64]
    dx1p = jnp.concatenate(
        [data_x[:, 64:120], jnp.zeros((_N, 8), jnp.float32)], axis=1)
    nidr = jnp.concatenate(
        [n_id0, jnp.zeros((_NP - _N,), jnp.int32)]).reshape(16, _NSUB, _RSUB)

    pidx = jnp.arange(_EP - _E, dtype=jnp.int32)
    ps = pidx % _N
    pd = _N + pidx % (_NP - _N)
    src0r = jnp.concatenate([edge_index0[0], ps]).reshape(-1, _GRP, _K)
    dst0r = jnp.concatenate([edge_index0[1], pd]).reshape(-1, _GRP, _K)
    s1p = jnp.concatenate([edge_index1[0], ps])
    srcb = jnp.concatenate([s1p, s1p + _N]).reshape(-1, _GRP, _K)
    dst1r = jnp.concatenate([edge_index1[1], pd]).reshape(-1, _GRP, _K)

    ones8 = jnp.ones((_K, 8), jnp.float32)
    z64 = jnp.zeros((_RSUB, 64), jnp.float32)
    z8 = jnp.zeros((_RSUB, 8), jnp.float32)
    z128 = jnp.zeros((_RSUB, 128), jnp.float32)

    o_acc, o_aux = _conv1(dx0, dx1p, data_node_index, nidr, emb_table,
                          src0r, dst0r, ones8, z64, z8)
    h1s = _mm1(o_acc.reshape(2, _NP, 64), o_aux.reshape(2, _NP, 8),
               W1, b1.reshape(1, _H))
    o3, o4 = _conv2(h1s.reshape(2 * _N, 128), srcb, dst1r, ones8, z128, z8)
    return _mm2(o3.reshape(2, _NP, 128), o4, W2, b2.reshape(1, _H),
                Wout, bout.reshape(1, _C))
